# Initial kernel scaffold; baseline (speedup 1.0000x reference)
import jax
import jax.numpy as jnp
from jax.experimental import pallas as pl


def kernel(x, Wr, br, We, be, Wo, bo, norm_w):
    raise NotImplementedError("write your pallas kernel here")


# speedup vs baseline: 1.8088x; 1.8088x over previous
import functools

import jax
import jax.numpy as jnp
from jax import lax
from jax.experimental import pallas as pl
from jax.experimental.pallas import tpu as pltpu
from jax.experimental.pallas import tpu_sc as plsc

D = 1024
E = 16
T = 2048
P = 2 * T
BM = 128
NBMAX = P // BM + E
NRMAX = NBMAX * BM
NW = 32
EPS = 1e-6
TB = 256


def _router_body(x_ref, wrt_ref, br_ref,
                 dest_ref, w0_ref, w1_ref, bex_ref, nbt_ref):
    xf = x_ref[...]
    logits = jnp.dot(xf, wrt_ref[...], preferred_element_type=jnp.float32)
    logits = logits + br_ref[...]
    lane = lax.broadcasted_iota(jnp.int32, (T, E), 1)
    m1 = jnp.max(logits, axis=1, keepdims=True)
    i1 = jnp.min(jnp.where(logits == m1, lane, E), axis=1, keepdims=True)
    l2 = jnp.where(lane == i1, -jnp.inf, logits)
    m2 = jnp.max(l2, axis=1, keepdims=True)
    i2 = jnp.min(jnp.where(l2 == m2, lane, E), axis=1, keepdims=True)
    w1 = 1.0 / (1.0 + jnp.exp(m2 - m1))
    w0_ref[...] = jnp.broadcast_to(w1, (T, 128))
    w1_ref[...] = jnp.broadcast_to(1.0 - w1, (T, 128))

    oh = jnp.concatenate([lane == i1, lane == i2], axis=0).astype(jnp.int32)
    csum = oh
    s = 1
    while s < P:
        csum = csum + jnp.concatenate(
            [jnp.zeros((s, E), jnp.int32), csum[:P - s]], axis=0)
        s *= 2
    counts = csum[P - 1:P, :]
    pc = ((counts + (BM - 1)) // BM) * BM
    ipc = pc
    s = 1
    while s < E:
        ipc = ipc + jnp.concatenate(
            [jnp.zeros((1, s), jnp.int32), ipc[:, :E - s]], axis=1)
        s *= 2
    poff = ipc - pc
    rank = jnp.sum(oh * csum, axis=1, keepdims=True) - 1
    base = jnp.sum(oh * jnp.broadcast_to(poff, (P, E)), axis=1, keepdims=True)
    dest_ref[...] = base + rank

    nblk_end = ipc // BM
    nbt = nblk_end[:, E - 1:E]
    nbt_ref[...] = nbt
    bi = lax.broadcasted_iota(jnp.int32, (NBMAX, E), 0)
    bic = jnp.minimum(bi, nbt - 1)
    bex_ref[...] = jnp.sum(
        (bic >= jnp.broadcast_to(nblk_end, (NBMAX, E))).astype(jnp.int32),
        axis=1, keepdims=True)


def _router(x_flat, wrt, br2):
    return pl.pallas_call(
        _router_body,
        out_shape=(
            jax.ShapeDtypeStruct((P, 1), jnp.int32),
            jax.ShapeDtypeStruct((T, 128), jnp.float32),
            jax.ShapeDtypeStruct((T, 128), jnp.float32),
            jax.ShapeDtypeStruct((NBMAX, 1), jnp.int32),
            jax.ShapeDtypeStruct((1, 1), jnp.int32),
        ),
    )(x_flat, wrt, br2)


def _sc_dispatch(x_flat, d_disp):
    mesh = plsc.VectorSubcoreMesh(core_axis_name="c", subcore_axis_name="s")

    @functools.partial(
        pl.kernel, mesh=mesh,
        out_type=jax.ShapeDtypeStruct((NRMAX, D), jnp.float32),
        scratch_types=[
            pltpu.VMEM((2, 64), jnp.int32),
            pltpu.VMEM((64, D), jnp.float32),
            pltpu.SemaphoreType.DMA,
        ],
    )
    def k(x_hbm, d_hbm, xs_hbm, idx_v, rows_v, sem):
        wid = lax.axis_index("s") * 2 + lax.axis_index("c")
        tok0 = (wid % 16) * 128
        pltpu.sync_copy(d_hbm.at[wid], idx_v)
        for c in range(2):
            pltpu.sync_copy(x_hbm.at[pl.ds(tok0 + c * 64, 64)], rows_v)
            pltpu.async_copy(rows_v, xs_hbm.at[idx_v.at[c]], sem).wait()

    return k(x_flat, d_disp)


def _gmm_body(meta_ref, xs_ref, we_ref, beb_ref, ys_ref):
    i = pl.program_id(0)

    @pl.when(i < meta_ref[0])
    def _():
        acc = lax.dot_general(
            xs_ref[...], we_ref[0],
            (((1,), (1,)), ((), ())),
            preferred_element_type=jnp.float32)
        acc = acc + beb_ref[0]
        ys_ref[...] = acc / (1.0 + jnp.exp(-acc))


def _gmm(meta, xs, We, be):
    grid_spec = pltpu.PrefetchScalarGridSpec(
        num_scalar_prefetch=1,
        grid=(NBMAX,),
        in_specs=[
            pl.BlockSpec((BM, D), lambda i, m: (i, 0)),
            pl.BlockSpec((1, D, D), lambda i, m: (m[1 + i], 0, 0)),
            pl.BlockSpec((1, 1, D), lambda i, m: (m[1 + i], 0, 0)),
        ],
        out_specs=pl.BlockSpec((BM, D), lambda i, m: (i, 0)),
    )
    return pl.pallas_call(
        _gmm_body,
        grid_spec=grid_spec,
        out_shape=jax.ShapeDtypeStruct((NRMAX, D), jnp.float32),
    )(meta, xs, We, be.reshape(E, 1, D))


def _sc_combine(ys, d0, d1):
    mesh = plsc.VectorSubcoreMesh(core_axis_name="c", subcore_axis_name="s")

    @functools.partial(
        pl.kernel, mesh=mesh,
        out_type=(
            jax.ShapeDtypeStruct((T, D), jnp.float32),
            jax.ShapeDtypeStruct((T, D), jnp.float32),
        ),
        scratch_types=[
            pltpu.VMEM((64,), jnp.int32),
            pltpu.VMEM((64,), jnp.int32),
            pltpu.VMEM((64, D), jnp.float32),
            pltpu.SemaphoreType.DMA,
        ],
    )
    def k(ys_hbm, d0_hbm, d1_hbm, g0_hbm, g1_hbm, i0_v, i1_v, rows_v, sem):
        wid = lax.axis_index("s") * 2 + lax.axis_index("c")
        tok0 = wid * 64
        pltpu.sync_copy(d0_hbm.at[wid], i0_v)
        pltpu.sync_copy(d1_hbm.at[wid], i1_v)
        pltpu.async_copy(ys_hbm.at[i0_v], rows_v, sem).wait()
        pltpu.sync_copy(rows_v, g0_hbm.at[pl.ds(tok0, 64)])
        pltpu.async_copy(ys_hbm.at[i1_v], rows_v, sem).wait()
        pltpu.sync_copy(rows_v, g1_hbm.at[pl.ds(tok0, 64)])

    return k(ys, d0, d1)


def _final_body(g0_ref, g1_ref, w0_ref, w1_ref, x_ref, wo_ref, bo_ref,
                nw_ref, o_ref):
    comb = g0_ref[...] * w0_ref[...][:, :1] + g1_ref[...] * w1_ref[...][:, :1]
    out = lax.dot_general(
        comb, wo_ref[...], (((1,), (1,)), ((), ())),
        preferred_element_type=jnp.float32) + bo_ref[...]
    h = x_ref[...] + out
    rms = lax.rsqrt(jnp.mean(h * h, axis=1, keepdims=True) + EPS)
    o_ref[...] = nw_ref[...] * (h * rms)


def _final(g0, g1, w0r, w1r, x_flat, Wo, bo2, nw2):
    return pl.pallas_call(
        _final_body,
        grid=(T // TB,),
        in_specs=[
            pl.BlockSpec((TB, D), lambda i: (i, 0)),
            pl.BlockSpec((TB, D), lambda i: (i, 0)),
            pl.BlockSpec((TB, 128), lambda i: (i, 0)),
            pl.BlockSpec((TB, 128), lambda i: (i, 0)),
            pl.BlockSpec((TB, D), lambda i: (i, 0)),
            pl.BlockSpec((D, D), lambda i: (0, 0)),
            pl.BlockSpec((1, D), lambda i: (0, 0)),
            pl.BlockSpec((1, D), lambda i: (0, 0)),
        ],
        out_specs=pl.BlockSpec((TB, D), lambda i: (i, 0)),
        out_shape=jax.ShapeDtypeStruct((T, D), jnp.float32),
    )(g0, g1, w0r, w1r, x_flat, Wo, bo2, nw2)


def kernel(x, Wr, br, We, be, Wo, bo, norm_w):
    x_flat = x.reshape(T, D)
    dest, w0r, w1r, bex, nbt = _router(x_flat, Wr.T, br.reshape(1, E))
    destf = dest.reshape(P)
    meta = jnp.concatenate([nbt.reshape(1), bex.reshape(NBMAX)])
    xs = _sc_dispatch(x_flat, destf.reshape(NW, 2, 64))
    ys = _gmm(meta, xs, We, be)
    g0, g1 = _sc_combine(ys, destf[:T].reshape(NW, 64),
                         destf[T:].reshape(NW, 64))
    out = _final(g0, g1, w0r, w1r, x_flat, Wo,
                 bo.reshape(1, D), norm_w.reshape(1, D))
    return out.reshape(1, T, D)
